# Initial kernel scaffold; baseline (speedup 1.0000x reference)
import jax
import jax.numpy as jnp
from jax.experimental import pallas as pl


def kernel(x, edge_index_rel0, edge_index_rel1, W1, b1, W2, b2, Wg, al, ar, bg, Wl, bl):
    raise NotImplementedError("write your pallas kernel here")


# speedup vs baseline: 8.7423x; 8.7423x over previous
import jax
import jax.numpy as jnp
from jax import lax
from jax.experimental import pallas as pl
from jax.experimental.pallas import tpu as pltpu
from jax.experimental.pallas import tpu_sc as plsc

_N = 10000
_D = 128
_H = 8
_FH = 16
_E = 160000
_NC = 2
_NS = 16
_C = 128
_NP = 10240
_NPK = _NP // 8
_CH = 80
_EPT = _CH * _C
_EP = _EPT * _NS
_RPT = _NP // _NS
_NB = 4
_NG = _CH // _NB

_NH = _NP // 2
_RPH = _NH // _NS
_NH3 = 3456
_RPH3 = _NH3 // _NS
_NP3 = 3 * _NH3

_mesh_cache = {}


def _mesh():
    if "m" not in _mesh_cache:
        _mesh_cache["m"] = plsc.VectorSubcoreMesh(
            core_axis_name="c", subcore_axis_name="s",
            num_cores=_NC, num_subcores=_NS,
        )
    return _mesh_cache["m"]


def _fill_zeros(ref, nrows, ncols):
    z = jnp.zeros((16,), jnp.float32)

    def body(r, _):
        for c in range(ncols // 16):
            ref[r, pl.ds(16 * c, 16)] = z
        return 0

    lax.fori_loop(0, nrows, body, 0)


def _shift_clamp_idx(src, dst, p, nh):
    lo = jnp.full((16,), p * nh, jnp.int32)
    hi = jnp.full((16,), nh, jnp.int32)

    def body(r, _):
        for c in range(_C // 16):
            v = src[r, pl.ds(16 * c, 16)] - lo
            ok = (v >= 0) & (v < hi)
            dst[r, pl.ds(16 * c, 16)] = jnp.where(ok, v, hi)
        return 0

    lax.fori_loop(0, _CH, body, 0)


def _zero_acc_half(acc, bufz, sid, nh, rph):
    off = 0
    while off < rph:
        n = min(_C, rph - off)
        pltpu.sync_copy(bufz.at[pl.ds(0, n)],
                        acc.at[pl.ds(sid * rph + off, n)])
        off += n

    @pl.when(sid == _NS - 1)
    def _():
        pltpu.sync_copy(bufz.at[pl.ds(0, 8)], acc.at[pl.ds(nh, 8)])


def _gather_body(tbl, idxh, out, idxv, bufz, gsem, wsem):
    cid = lax.axis_index("c")
    sid = lax.axis_index("s")
    pltpu.sync_copy(idxh.at[cid].at[pl.ds(sid * _CH, _CH)], idxv)
    tr = tbl.at[cid]
    for g in range(_NG):
        hs = []
        for b in range(_NB):
            j = g * _NB + b
            hs.append(pltpu.async_copy(tr.at[idxv.at[j]], bufz.at[b],
                                       gsem.at[b]))
        ws = []
        for b in range(_NB):
            j = g * _NB + b
            hs[b].wait()
            ws.append(pltpu.async_copy(
                bufz.at[b],
                out.at[cid].at[pl.ds(sid * _EPT + j * _C, _C)],
                wsem.at[b]))
        for w in ws:
            w.wait()


def _gather_call(rows):
  return pl.kernel(
    _gather_body,
    out_type=jax.ShapeDtypeStruct((_NC, _EP, _D), jnp.float32),
    mesh=_mesh(),
    scratch_types=[
        pltpu.VMEM((_CH, _C), jnp.int32),
        pltpu.VMEM((_NB, _C, _D), jnp.float32),
        pltpu.SemaphoreType.DMA((_NB,)),
        pltpu.SemaphoreType.DMA((_NB,)),
    ],
    name=f"sc_gather_{rows}",
  )


def _pscatter_body(vals, idxh, out, idxv, bufs, lsem, ssem, acc):
    cid = lax.axis_index("c")
    sid = lax.axis_index("s")
    pltpu.sync_copy(idxh.at[cid].at[pl.ds(sid * _CH, _CH)], idxv)
    vr = vals.at[cid]
    _fill_zeros(bufs.at[0], _C, _D)
    pltpu.sync_copy(bufs.at[0].at[pl.ds(0, _NPK // _NS)],
                    acc.at[pl.ds(sid * (_NPK // _NS), _NPK // _NS)])
    plsc.subcore_barrier()
    for g in range(_NG):
        lh = []
        for b in range(_NB):
            j = g * _NB + b
            lh.append(pltpu.async_copy(
                vr.at[pl.ds(sid * _EPT + j * _C, _C)],
                bufs.at[b], lsem.at[b]))
        sh = []
        for b in range(_NB):
            j = g * _NB + b
            lh[b].wait()
            sh.append(pltpu.async_copy(bufs.at[b], acc.at[idxv.at[j]],
                                       ssem.at[b], add=True))
        for h in sh:
            h.wait()
    plsc.subcore_barrier()
    pltpu.sync_copy(acc.at[pl.ds(sid * (_NPK // _NS), _NPK // _NS)],
                    out.at[cid].at[pl.ds(sid * (_NPK // _NS), _NPK // _NS)])


def _pscatter_call():
  return pl.kernel(
    _pscatter_body,
    out_type=jax.ShapeDtypeStruct((_NC, _NPK, _D), jnp.float32),
    mesh=_mesh(),
    scratch_types=[
        pltpu.VMEM((_CH, _C), jnp.int32),
        pltpu.VMEM((_NB, _C, _D), jnp.float32),
        pltpu.SemaphoreType.DMA((_NB,)),
        pltpu.SemaphoreType.DMA((_NB,)),
        pltpu.VMEM_SHARED((_NPK, _D), jnp.float32),
    ],
  )


def _gcn_body(tblf, edges, out, idxs, idxd, idxw, bufs, gsem, ssem, acc):
    cid = lax.axis_index("c")
    sid = lax.axis_index("s")
    pltpu.sync_copy(edges.at[cid].at[pl.ds(sid * _CH, _CH)], idxs)
    pltpu.sync_copy(edges.at[cid].at[pl.ds(_NS * _CH + sid * _CH, _CH)],
                    idxd)
    tblr = tblf.at[cid]
    _fill_zeros(bufs.at[0], _C, _D)
    for p in range(3):
        _shift_clamp_idx(idxd, idxw, p, _NH3)
        _zero_acc_half(acc, bufs.at[0], sid, _NH3, _RPH3)
        plsc.subcore_barrier()
        for g in range(_NG):
            gh = []
            for b in range(_NB):
                j = g * _NB + b
                gh.append(pltpu.async_copy(tblr.at[idxs.at[j]], bufs.at[b],
                                           gsem.at[b]))
            sh = []
            for b in range(_NB):
                j = g * _NB + b
                gh[b].wait()
                sh.append(pltpu.async_copy(bufs.at[b], acc.at[idxw.at[j]],
                                           ssem.at[b], add=True))
            for h in sh:
                h.wait()
        plsc.subcore_barrier()
        pltpu.sync_copy(acc.at[pl.ds(sid * _RPH3, _RPH3)],
                        out.at[cid].at[pl.ds(p * _NH3 + sid * _RPH3, _RPH3)])
        plsc.subcore_barrier()
        _fill_zeros(bufs.at[0], _C, _D)


def _gcn_call():
  return pl.kernel(
    _gcn_body,
    out_type=jax.ShapeDtypeStruct((_NC, _NP3, 128), jnp.float32),
    mesh=_mesh(),
    scratch_types=[
        pltpu.VMEM((_CH, _C), jnp.int32),
        pltpu.VMEM((_CH, _C), jnp.int32),
        pltpu.VMEM((_CH, _C), jnp.int32),
        pltpu.VMEM((_NB, _C, _D), jnp.float32),
        pltpu.SemaphoreType.DMA((_NB,)),
        pltpu.SemaphoreType.DMA((_NB,)),
        pltpu.VMEM_SHARED((_NH3 + 8, _D), jnp.float32),
    ],
  )


def _gat3u_body(u, dsts, out_u, idxd, idxw, bufs, lsem, ssem, acc_u):
    cid = lax.axis_index("c")
    sid = lax.axis_index("s")
    pltpu.sync_copy(dsts.at[cid].at[pl.ds(sid * _CH, _CH)], idxd)
    ur = u.at[cid]
    _fill_zeros(bufs.at[0], _C, _D)
    for p in range(2):
        _shift_clamp_idx(idxd, idxw, p, _NH)
        _zero_acc_half(acc_u, bufs.at[0], sid, _NH, _RPH)
        plsc.subcore_barrier()
        for g in range(_NG):
            lh = []
            for b in range(_NB):
                j = g * _NB + b
                lh.append(pltpu.async_copy(
                    ur.at[pl.ds(sid * _EPT + j * _C, _C)],
                    bufs.at[b], lsem.at[b]))
            sh = []
            for b in range(_NB):
                j = g * _NB + b
                lh[b].wait()
                sh.append(pltpu.async_copy(bufs.at[b], acc_u.at[idxw.at[j]],
                                           ssem.at[b], add=True))
            for h in sh:
                h.wait()
        plsc.subcore_barrier()
        pltpu.sync_copy(acc_u.at[pl.ds(sid * _RPH, _RPH)],
                        out_u.at[cid].at[pl.ds(p * _NH + sid * _RPH, _RPH)])
        plsc.subcore_barrier()
        _fill_zeros(bufs.at[0], _C, _D)


def _gat3u_call():
  return pl.kernel(
    _gat3u_body,
    out_type=jax.ShapeDtypeStruct((_NC, _NP, _D), jnp.float32),
    mesh=_mesh(),
    scratch_types=[
        pltpu.VMEM((_CH, _C), jnp.int32),
        pltpu.VMEM((_CH, _C), jnp.int32),
        pltpu.VMEM((_NB, _C, _D), jnp.float32),
        pltpu.SemaphoreType.DMA((_NB,)),
        pltpu.SemaphoreType.DMA((_NB,)),
        pltpu.VMEM_SHARED((_NH + 8, _D), jnp.float32),
    ],
  )


def _pad_edges(idx):
    pad = jnp.full((_EP - _E,), _N, jnp.int32)
    return jnp.concatenate([idx.astype(jnp.int32), pad]).reshape(_NS * _CH, _C)


def _pad_rows(a):
    widths = [(0, 0)] * (a.ndim - 2) + [(0, _NP - _N), (0, 0)]
    return jnp.pad(a, widths)


def _onehot_slot(idx):
    return (jnp.arange(128, dtype=jnp.int32)[None, None, :]
            == ((idx % 8) * 16)[:, :, None]).astype(jnp.float32)


def kernel(x, edge_index_rel0, edge_index_rel1, W1, b1, W2, b2, Wg, al, ar,
           bg, Wl, bl):
    srcs = jnp.stack([_pad_edges(edge_index_rel0[0]),
                      _pad_edges(edge_index_rel1[0])])
    dsts = jnp.stack([_pad_edges(edge_index_rel0[1]),
                      _pad_edges(edge_index_rel1[1])])
    edges = jnp.concatenate([srcs, dsts], axis=1)
    srcp = srcs.reshape(_NC, _NS * _CH, _C) >> 3
    dstp = dsts.reshape(_NC, _NS * _CH, _C) >> 3

    sflat = srcs.reshape(_NC, _EP)
    dflat = dsts.reshape(_NC, _EP)
    dego_pk = _pscatter_call()(_onehot_slot(sflat), srcp)
    degi_pk = _pscatter_call()(_onehot_slot(dflat), dstp)
    deg_o = dego_pk.reshape(_NC, _NP, 16)[:, :_N, 0]
    deg_i = degi_pk.reshape(_NC, _NP, 16)[:, :_N, 0]
    ns = jax.lax.rsqrt(jnp.clip(deg_o, 1.0))
    nd = jax.lax.rsqrt(jnp.clip(deg_i, 1.0))

    h = x
    for (W, b) in ((W1, b1), (W2, b2)):
        tbl = _pad_rows(h[None] * ns[:, :, None])
        agg = _gcn_call()(tbl, edges)[:, :_N]
        h0 = agg[0] * nd[0][:, None] @ W[0] + b[0]
        h1 = agg[1] * nd[1][:, None] @ W[1] + b[1]
        h = jax.nn.leaky_relu((h0 + h1) * 0.5, 0.01)

    z = jnp.stack([h @ Wg[0], h @ Wg[1]])
    zr = z.reshape(_NC, _N, _H, _FH)
    er = jnp.sum(zr * ar[:, None], axis=-1)
    pad8 = jnp.zeros((_NC, _N, 8), jnp.float32)
    er16 = _pad_rows(jnp.concatenate([er, pad8], axis=-1))
    erp = er16.reshape(_NC, _NPK, 128)
    zp = _pad_rows(z)

    zz = _gather_call(_NP)(zp, srcs)
    er_lines = _gather_call(_NPK)(erp, dstp)
    er_g = jnp.take_along_axis(
        er_lines.reshape(_NC, _EP, 8, 16),
        (dflat % 8)[:, :, None, None], axis=2)[:, :, 0, :8]
    el_g = jnp.sum(zz.reshape(_NC, _EP, _H, _FH) * al[:, None], axis=-1)
    e = jax.nn.leaky_relu(el_g + er_g, 0.2)
    ee = jnp.exp(e)
    u = zz * jnp.repeat(ee, _FH, axis=-1)

    acc_u = _gat3u_call()(u, dsts)
    ee16 = jnp.concatenate(
        [ee, jnp.zeros((_NC, _EP, 8), jnp.float32)], axis=-1)
    vals_d = (ee16[:, :, None, :]
              * (jnp.arange(8, dtype=jnp.int32)[None, None, :, None]
                 == (dflat % 8)[:, :, None, None])).reshape(_NC, _EP, 128)
    den_pk = _pscatter_call()(vals_d, dstp)
    denom = jnp.clip(den_pk.reshape(_NC, _NP, 16)[:, :_N, :_H], 1e-9)
    gat = acc_u[:, :_N].reshape(_NC, _N, _H, _FH) / denom[..., None] \
        + bg.reshape(_NC, 1, _H, _FH)
    g = ((gat[0] + gat[1]) * 0.5).reshape(_N, _H * _FH)
    return g @ Wl + bl
